# speedup vs baseline: 17.6853x; 17.6853x over previous
import functools

import jax
import jax.numpy as jnp
from jax import lax
from jax.experimental import pallas as pl
from jax.experimental.pallas import tpu as pltpu
from jax.experimental.pallas import tpu_sc as plsc

_GATHER_WINDOW = 128


def _sc_gather(emb2d, flat_idx):
    num_indices = flat_idx.shape[1]
    d = emb2d.shape[1]
    mesh = plsc.VectorSubcoreMesh(core_axis_name="core", subcore_axis_name="subcore")

    @functools.partial(
        pl.kernel,
        out_type=jax.ShapeDtypeStruct((num_indices, d), emb2d.dtype),
        mesh=mesh,
    )
    def gather_kernel(x_hbm, i_hbm, o_hbm):
        def body(i_vmem, o_vmem):
            pltpu.sync_copy(x_hbm.at[i_vmem.at[0]], o_vmem)

        pltpu.emit_pipeline(
            body,
            grid=(num_indices // _GATHER_WINDOW,),
            in_specs=[
                pl.BlockSpec((1, _GATHER_WINDOW), index_map=lambda i: (0, i))
            ],
            out_specs=[
                pl.BlockSpec((_GATHER_WINDOW, d), index_map=lambda i: (i, 0))
            ],
            core_axis_name=("core", "subcore"),
            dimension_semantics=(pltpu.PARALLEL,),
        )(i_hbm, o_hbm)

    return gather_kernel(emb2d, flat_idx)


def _decode_body(rb, k, cand_ref, w1_ref, b1_ref, w2_ref, b2_ref, w3_ref,
                 b3_ref, mask_ref, gum_ref, logp_ref, act_ref):
    x = cand_ref[...]
    h = jnp.tanh(jnp.dot(x, w1_ref[...]) + b1_ref[...])
    h = jnp.tanh(jnp.dot(h, w2_ref[...]) + b2_ref[...])
    logits = jnp.dot(h, w3_ref[...]) + b3_ref[...]
    logits = logits.reshape(rb, k)
    mask = mask_ref[...]
    neg_inf = jnp.float32(-jnp.inf)
    logits = jnp.where(mask, logits, neg_inf)
    xm = jnp.max(logits, axis=1, keepdims=True)
    shifted = logits - xm
    lse = jnp.log(jnp.sum(jnp.exp(shifted), axis=1, keepdims=True))
    logp = shifted - lse
    logp_ref[...] = logp
    gumbel = -jnp.log(-jnp.log(gum_ref[...]))
    keys = jnp.where(mask, logp + gumbel, neg_inf)
    km = jnp.max(keys, axis=1, keepdims=True)
    iota = lax.broadcasted_iota(jnp.int32, (rb, k), 1)
    first_max = jnp.min(jnp.where(keys == km, iota, k), axis=1)
    act_ref[...] = first_max[:, None]


def _tc_decode(cand2d, W1, b1, W2, b2, W3, b3, action_mask, gumbel_u, rb):
    b, k = action_mask.shape
    d = cand2d.shape[1]
    grid = (b // rb,)
    body = functools.partial(_decode_body, rb, k)
    return pl.pallas_call(
        body,
        grid=grid,
        in_specs=[
            pl.BlockSpec((rb * k, d), lambda i: (i, 0)),
            pl.BlockSpec((d, d), lambda i: (0, 0)),
            pl.BlockSpec((1, d), lambda i: (0, 0)),
            pl.BlockSpec((d, d), lambda i: (0, 0)),
            pl.BlockSpec((1, d), lambda i: (0, 0)),
            pl.BlockSpec((d, 1), lambda i: (0, 0)),
            pl.BlockSpec((1, 1), lambda i: (0, 0)),
            pl.BlockSpec((rb, k), lambda i: (i, 0)),
            pl.BlockSpec((rb, k), lambda i: (i, 0)),
        ],
        out_specs=[
            pl.BlockSpec((rb, k), lambda i: (i, 0)),
            pl.BlockSpec((rb, 1), lambda i: (i, 0)),
        ],
        out_shape=[
            jax.ShapeDtypeStruct((b, k), jnp.float32),
            jax.ShapeDtypeStruct((b, 1), jnp.int32),
        ],
        compiler_params=pltpu.CompilerParams(
            dimension_semantics=("parallel",),
        ),
    )(cand2d, W1, b1.reshape(1, d), W2, b2.reshape(1, d), W3,
      b3.reshape(1, 1), action_mask, gumbel_u)


def kernel(embeddings, gumbel_u, W1, b1, W2, b2, W3, b3, next_op, action_mask):
    b, n, d = embeddings.shape
    k = next_op.shape[1]
    emb2d = embeddings.reshape(b * n, d)
    flat_idx = (
        next_op.astype(jnp.int32)
        + (jnp.arange(b, dtype=jnp.int32) * n)[:, None]
    ).reshape(1, b * k)
    cand2d = _sc_gather(emb2d, flat_idx)
    log_p, actions = _tc_decode(
        cand2d, W1, b1, W2, b2, W3, b3, action_mask, gumbel_u, rb=8
    )
    return (log_p, actions.reshape(b))


# speedup vs baseline: 18.5739x; 1.0502x over previous
import functools

import jax
import jax.numpy as jnp
from jax import lax
from jax.experimental import pallas as pl
from jax.experimental.pallas import tpu as pltpu
from jax.experimental.pallas import tpu_sc as plsc

_GATHER_WINDOW = 128


def _sc_gather(emb2d, flat_idx):
    num_indices = flat_idx.shape[1]
    d = emb2d.shape[1]
    mesh = plsc.VectorSubcoreMesh(core_axis_name="core", subcore_axis_name="subcore")

    @functools.partial(
        pl.kernel,
        out_type=jax.ShapeDtypeStruct((num_indices, d), emb2d.dtype),
        mesh=mesh,
    )
    def gather_kernel(x_hbm, i_hbm, o_hbm):
        def body(i_vmem, o_vmem):
            pltpu.sync_copy(x_hbm.at[i_vmem.at[0]], o_vmem)

        pltpu.emit_pipeline(
            body,
            grid=(num_indices // _GATHER_WINDOW,),
            in_specs=[
                pl.BlockSpec((1, _GATHER_WINDOW), index_map=lambda i: (0, i))
            ],
            out_specs=[
                pl.BlockSpec((_GATHER_WINDOW, d), index_map=lambda i: (i, 0))
            ],
            core_axis_name=("core", "subcore"),
            dimension_semantics=(pltpu.PARALLEL,),
        )(i_hbm, o_hbm)

    return gather_kernel(emb2d, flat_idx)


def _decode_body(rb, k, cand_ref, w1_ref, b1_ref, w2_ref, b2_ref, w3_ref,
                 b3_ref, mask_ref, gum_ref, logp_ref, act_ref):
    x = cand_ref[...]
    h = jnp.tanh(jnp.dot(x, w1_ref[...]) + b1_ref[...])
    h = jnp.tanh(jnp.dot(h, w2_ref[...]) + b2_ref[...])
    logits = jnp.dot(h, w3_ref[...]) + b3_ref[...]
    logits = logits.reshape(rb, k)
    mask = mask_ref[...]
    neg_inf = jnp.float32(-jnp.inf)
    logits = jnp.where(mask, logits, neg_inf)
    xm = jnp.max(logits, axis=1, keepdims=True)
    shifted = logits - xm
    lse = jnp.log(jnp.sum(jnp.exp(shifted), axis=1, keepdims=True))
    logp = shifted - lse
    logp_ref[...] = logp
    gumbel = -jnp.log(-jnp.log(gum_ref[...]))
    keys = jnp.where(mask, logp + gumbel, neg_inf)
    km = jnp.max(keys, axis=1, keepdims=True)
    iota = lax.broadcasted_iota(jnp.int32, (rb, k), 1)
    first_max = jnp.min(jnp.where(keys == km, iota, k), axis=1)
    act_ref[...] = first_max[:, None]


def _tc_decode(cand2d, W1, b1, W2, b2, W3, b3, action_mask, gumbel_u, rb):
    b, k = action_mask.shape
    d = cand2d.shape[1]
    grid = (b // rb,)
    body = functools.partial(_decode_body, rb, k)
    return pl.pallas_call(
        body,
        grid=grid,
        in_specs=[
            pl.BlockSpec((rb * k, d), lambda i: (i, 0)),
            pl.BlockSpec((d, d), lambda i: (0, 0)),
            pl.BlockSpec((1, d), lambda i: (0, 0)),
            pl.BlockSpec((d, d), lambda i: (0, 0)),
            pl.BlockSpec((1, d), lambda i: (0, 0)),
            pl.BlockSpec((d, 1), lambda i: (0, 0)),
            pl.BlockSpec((1, 1), lambda i: (0, 0)),
            pl.BlockSpec((rb, k), lambda i: (i, 0)),
            pl.BlockSpec((rb, k), lambda i: (i, 0)),
        ],
        out_specs=[
            pl.BlockSpec((rb, k), lambda i: (i, 0)),
            pl.BlockSpec((rb, 1), lambda i: (i, 0)),
        ],
        out_shape=[
            jax.ShapeDtypeStruct((b, k), jnp.float32),
            jax.ShapeDtypeStruct((b, 1), jnp.int32),
        ],
        compiler_params=pltpu.CompilerParams(
            dimension_semantics=("parallel",),
        ),
    )(cand2d, W1, b1.reshape(1, d), W2, b2.reshape(1, d), W3,
      b3.reshape(1, 1), action_mask, gumbel_u)


def kernel(embeddings, gumbel_u, W1, b1, W2, b2, W3, b3, next_op, action_mask):
    b, n, d = embeddings.shape
    k = next_op.shape[1]
    emb2d = embeddings.reshape(b * n, d)
    flat_idx = (
        next_op.astype(jnp.int32)
        + (jnp.arange(b, dtype=jnp.int32) * n)[:, None]
    )
    n_chunks = 4
    bc = b // n_chunks
    logps, acts = [], []
    for c in range(n_chunks):
        sl = slice(c * bc, (c + 1) * bc)
        cand_c = _sc_gather(emb2d, flat_idx[sl].reshape(1, bc * k))
        lp, ac = _tc_decode(
            cand_c, W1, b1, W2, b2, W3, b3,
            action_mask[sl], gumbel_u[sl], rb=8,
        )
        logps.append(lp)
        acts.append(ac)
    log_p = jnp.concatenate(logps, axis=0)
    actions = jnp.concatenate(acts, axis=0).reshape(b)
    return (log_p, actions)
